# baseline (speedup 1.0000x reference)
import functools

import jax
import jax.numpy as jnp
from jax.experimental import pallas as pl
from jax.experimental.pallas import tpu as pltpu
from jax.experimental.pallas import tpu_sc as plsc

_LANES = 16


def _last_token_gather(x_hbm, idx_hbm, out_hbm, idx_s, sem):
    B, C = out_hbm.shape

    pltpu.sync_copy(idx_hbm, idx_s)
    copies = []
    for b in range(B):
        copies.append(
            pltpu.make_async_copy(
                x_hbm.at[pl.ds(idx_s[b], 1)], out_hbm.at[pl.ds(b, 1)], sem
            )
        )
    for cp in copies:
        cp.start()
    for cp in copies:
        cp.wait()


def kernel(x, lengths):
    B, T, C = x.shape
    x_flat = x.reshape(B * T, C)
    lane = jnp.arange(_LANES, dtype=jnp.int32)
    li = jnp.maximum(lengths.astype(jnp.int32) - 1, 0)
    idx = jnp.where(lane < B, jnp.pad(li, (0, _LANES - B)) + lane * T, 0)

    mesh = plsc.ScalarSubcoreMesh(axis_name="c", num_cores=1)
    run = functools.partial(
        pl.kernel,
        out_type=jax.ShapeDtypeStruct((B, C), x.dtype),
        mesh=mesh,
        scratch_types=[
            pltpu.SMEM((_LANES,), jnp.int32),
            pltpu.SemaphoreType.DMA,
        ],
    )(_last_token_gather)
    return run(x_flat, idx)
